# baseline (speedup 1.0000x reference)
import jax
import jax.numpy as jnp
from jax import lax
from jax.experimental import pallas as pl
from jax.experimental.pallas import tpu as pltpu, tpu_sc as plsc

NUM_CORES = 2
NUM_SUBCORES = 16
LANES = 16
NW = NUM_CORES * NUM_SUBCORES

B = 16384
S = B // NW
NF, FD = 26, 1000
NC, CD = 8, 100
FP = NF // 2
CP = NC // 2
CHUNKS = S // LANES


def _body(user_h, item_h, feat_h, ctx_h, gb_h, ub_h, ib_h, ftab_h, ctab_h,
          out_h,
          uidx, iidx, urows, irows, fidx, cidx, ftab, ctab, gbv, outv,
          sem_u, sem_i, sem_s, sem_t):
    wid = lax.axis_index("s") * NUM_CORES + lax.axis_index("c")
    base = wid * S

    with jax.named_scope("stage_issue"):
        cu0 = pltpu.async_copy(user_h.at[pl.ds(base, S)], uidx, sem_u)
        ci0 = pltpu.async_copy(item_h.at[pl.ds(base, S)], iidx, sem_i)
        c1 = pltpu.async_copy(ftab_h, ftab, sem_t)
        c2 = pltpu.async_copy(ctab_h, ctab, sem_s)
        c3 = pltpu.async_copy(feat_h.at[:, pl.ds(base, S)], fidx, sem_s)
        c4 = pltpu.async_copy(ctx_h.at[:, pl.ds(base, S)], cidx, sem_s)
        c5 = pltpu.async_copy(gb_h, gbv, sem_s)
        cu0.wait()
        ci0.wait()
        cu = pltpu.async_copy(ub_h.at[uidx], urows, sem_u)
        ci = pltpu.async_copy(ib_h.at[iidx], irows, sem_i)
    with jax.named_scope("wait_slab"):
        c3.wait()
        c4.wait()
        c5.wait()
        c2.wait()
    with jax.named_scope("wait_ui"):
        cu.wait()
        ci.wait()

    gvec = gbv[...]

    scope1 = jax.named_scope("pass1_ctx")
    scope1.__enter__()

    @plsc.parallel_loop(0, CHUNKS, step=1, unroll=2)
    def chunk1(k):
        o = pl.ds(k * LANES, LANES)
        acc = gvec + urows[o] + irows[o]
        for c in range(NC):
            p, j = divmod(c, 2)
            vals = lax.shift_right_logical(cidx[p, o], 7 * j) & 0x7F
            row = jnp.full((LANES,), c, jnp.int32)
            acc = acc + plsc.load_gather(ctab, [row, vals])
        outv[o] = acc
    scope1.__exit__(None, None, None)

    with jax.named_scope("wait_ftab"):
        c1.wait()

    scope2 = jax.named_scope("pass2_feat")
    scope2.__enter__()

    @plsc.parallel_loop(0, CHUNKS, step=1, unroll=2)
    def chunk2(k):
        o = pl.ds(k * LANES, LANES)
        acc = outv[o]
        for f in range(NF):
            p, j = divmod(f, 2)
            vals = lax.shift_right_logical(fidx[p, o], 10 * j) & 0x3FF
            row = jnp.full((LANES,), f, jnp.int32)
            acc = acc + plsc.load_gather(ftab, [row, vals])
        outv[o] = acc
    scope2.__exit__(None, None, None)
    with jax.named_scope("writeback"):
        pltpu.sync_copy(outv, out_h.at[pl.ds(base, S)])


def kernel(user, item, features, contexts, global_bias, user_bias, item_bias,
           feat_bias, ctx_bias, feat_offsets, ctx_offsets):
    del feat_offsets, ctx_offsets
    feat_i = features.astype(jnp.int32)
    fpack = (feat_i[:, 0::2] | (feat_i[:, 1::2] << 10)).T
    ctx_i = contexts.astype(jnp.int32)
    cpack = (ctx_i[:, 0::2] | (ctx_i[:, 1::2] << 7)).T
    ftab = feat_bias.reshape(NF, FD)
    ctab = ctx_bias.reshape(NC, CD)
    ub = user_bias.reshape(-1)
    ib = item_bias.reshape(-1)
    gb16 = jnp.broadcast_to(global_bias, (LANES,))

    run = pl.kernel(
        _body,
        out_type=jax.ShapeDtypeStruct((B,), jnp.float32),
        mesh=plsc.VectorSubcoreMesh(core_axis_name="c", subcore_axis_name="s"),
        compiler_params=pltpu.CompilerParams(needs_layout_passes=False),
        scratch_types=[
            pltpu.VMEM((S,), jnp.int32),
            pltpu.VMEM((S,), jnp.int32),
            pltpu.VMEM((S,), jnp.float32),
            pltpu.VMEM((S,), jnp.float32),
            pltpu.VMEM((FP, S), jnp.int32),
            pltpu.VMEM((CP, S), jnp.int32),
            pltpu.VMEM((NF, FD), jnp.float32),
            pltpu.VMEM((NC, CD), jnp.float32),
            pltpu.VMEM((LANES,), jnp.float32),
            pltpu.VMEM((S,), jnp.float32),
            pltpu.SemaphoreType.DMA,
            pltpu.SemaphoreType.DMA,
            pltpu.SemaphoreType.DMA,
            pltpu.SemaphoreType.DMA,
        ],
    )
    return run(user.astype(jnp.int32), item.astype(jnp.int32), fpack,
               cpack, gb16, ub, ib, ftab, ctab)


# speedup vs baseline: 1.3206x; 1.3206x over previous
import jax
import jax.numpy as jnp
from jax import lax
from jax.experimental import pallas as pl
from jax.experimental.pallas import tpu as pltpu, tpu_sc as plsc

NUM_CORES = 2
NUM_SUBCORES = 16
LANES = 16
NW = NUM_CORES * NUM_SUBCORES

B = 16384
S = B // NW
NF, FD = 26, 1000
NC, CD = 8, 100
CHUNKS = S // LANES


def _body(user_h, item_h, feat_h, ctx_h, gb_h, ub_h, ib_h, ftab_h, ctab_h,
          out_h,
          uidx, iidx, urows, irows, fidx, cidx, ftab, ctab, gbv, outv,
          sem_u, sem_i, sem_s, sem_t):
    wid = lax.axis_index("s") * NUM_CORES + lax.axis_index("c")
    base = wid * S

    with jax.named_scope("stage_issue"):
        cu0 = pltpu.async_copy(user_h.at[pl.ds(base, S)], uidx, sem_u)
        ci0 = pltpu.async_copy(item_h.at[pl.ds(base, S)], iidx, sem_i)
        c1 = pltpu.async_copy(ftab_h, ftab, sem_t)
        c2 = pltpu.async_copy(ctab_h, ctab, sem_s)
        c3 = pltpu.async_copy(feat_h.at[:, pl.ds(base, S)], fidx, sem_s)
        c4 = pltpu.async_copy(ctx_h.at[:, pl.ds(base, S)], cidx, sem_s)
        c5 = pltpu.async_copy(gb_h, gbv, sem_s)
        cu0.wait()
        ci0.wait()
        cu = pltpu.async_copy(ub_h.at[uidx], urows, sem_u)
        ci = pltpu.async_copy(ib_h.at[iidx], irows, sem_i)
    with jax.named_scope("wait_slab"):
        c3.wait()
        c4.wait()
        c5.wait()
        c2.wait()
    gvec = gbv[...]

    scope1 = jax.named_scope("pass1_ctx")
    scope1.__enter__()

    @plsc.parallel_loop(0, CHUNKS, step=1, unroll=2)
    def chunk1(k):
        o = pl.ds(k * LANES, LANES)
        acc = gvec + plsc.load_gather(ctab, [jnp.zeros((LANES,), jnp.int32),
                                             cidx[0, o]])
        for c in range(1, NC):
            row = jnp.full((LANES,), c, jnp.int32)
            acc = acc + plsc.load_gather(ctab, [row, cidx[c, o]])
        outv[o] = acc
    scope1.__exit__(None, None, None)

    with jax.named_scope("wait_ui2"):
        cu.wait()
        ci.wait()
    with jax.named_scope("wait_ftab"):
        c1.wait()

    scope2 = jax.named_scope("pass2_feat")
    scope2.__enter__()

    @plsc.parallel_loop(0, CHUNKS, step=1, unroll=2)
    def chunk2(k):
        o = pl.ds(k * LANES, LANES)
        acc = outv[o] + urows[o] + irows[o]
        for f in range(NF):
            row = jnp.full((LANES,), f, jnp.int32)
            acc = acc + plsc.load_gather(ftab, [row, fidx[f, o]])
        outv[o] = acc
    scope2.__exit__(None, None, None)
    with jax.named_scope("writeback"):
        pltpu.sync_copy(outv, out_h.at[pl.ds(base, S)])


def kernel(user, item, features, contexts, global_bias, user_bias, item_bias,
           feat_bias, ctx_bias, feat_offsets, ctx_offsets):
    del feat_offsets, ctx_offsets
    featT = features.astype(jnp.int32).T
    ctxT = contexts.astype(jnp.int32).T
    ftab = feat_bias.reshape(NF, FD)
    ctab = ctx_bias.reshape(NC, CD)
    ub = user_bias.reshape(-1)
    ib = item_bias.reshape(-1)
    gb16 = jnp.broadcast_to(global_bias, (LANES,))

    run = pl.kernel(
        _body,
        out_type=jax.ShapeDtypeStruct((B,), jnp.float32),
        mesh=plsc.VectorSubcoreMesh(core_axis_name="c", subcore_axis_name="s"),
        compiler_params=pltpu.CompilerParams(needs_layout_passes=False),
        scratch_types=[
            pltpu.VMEM((S,), jnp.int32),
            pltpu.VMEM((S,), jnp.int32),
            pltpu.VMEM((S,), jnp.float32),
            pltpu.VMEM((S,), jnp.float32),
            pltpu.VMEM((NF, S), jnp.int32),
            pltpu.VMEM((NC, S), jnp.int32),
            pltpu.VMEM((NF, FD), jnp.float32),
            pltpu.VMEM((NC, CD), jnp.float32),
            pltpu.VMEM((LANES,), jnp.float32),
            pltpu.VMEM((S,), jnp.float32),
            pltpu.SemaphoreType.DMA,
            pltpu.SemaphoreType.DMA,
            pltpu.SemaphoreType.DMA,
            pltpu.SemaphoreType.DMA,
        ],
    )
    return run(user.astype(jnp.int32), item.astype(jnp.int32), featT,
               ctxT, gb16, ub, ib, ftab, ctab)
